# baseline (speedup 1.0000x reference)
import jax
import jax.numpy as jnp
from jax import lax
from jax.experimental import pallas as pl
from jax.experimental.pallas import tpu as pltpu
from jax.experimental.pallas import tpu_sc as plsc

NUM_EMB = 1201
EMB_DIM = 32
VOCAB = 1000
B, T, N = 1024, 50, 20
NUM_WORDS = B * T
NC, NS = 2, 16
NW = NC * NS
NSLICE = 2
TSL = T // NSLICE
WORDS_SL = B * TSL
WPW = WORDS_SL // NW
CH = 80
NCHUNK = WPW // CH
ROWS = CH * N
NGATH = 20
GSZ = ROWS // NGATH
PADD = 128


def _sc_body(xf_hbm, emb_hbm, word_hbm, idx_v, rows_v, out_v, sem, sem_i, sem_o):
    wid = lax.axis_index("s") * NC + lax.axis_index("c")
    wbase = wid * WPW
    c16 = jnp.full((16,), 16, jnp.uint32)
    cmask = jnp.full((16,), 0xFFFF0000, jnp.uint32)

    def _stage_idx(c, buf):
        return pltpu.async_copy(
            xf_hbm.at[pl.ds((wbase + c * CH) * N, ROWS)], idx_v.at[buf], sem_i
        )

    def _wait_idx(buf):
        pltpu.make_async_copy(
            xf_hbm.at[pl.ds(0, ROWS)], idx_v.at[buf], sem_i
        ).wait()

    def _fire_gathers(buf):
        for k in range(NGATH):
            pltpu.async_copy(
                emb_hbm.at[idx_v.at[buf, pl.ds(k * GSZ, GSZ)]],
                rows_v.at[buf, pl.ds(k * GSZ, GSZ)],
                sem,
            )

    def _wait_gathers(buf):
        for k in range(NGATH):
            pltpu.make_async_copy(
                emb_hbm.at[idx_v.at[buf, pl.ds(k * GSZ, GSZ)]],
                rows_v.at[buf, pl.ds(k * GSZ, GSZ)],
                sem,
            ).wait()

    zeros16 = jnp.zeros((16,), jnp.float32)

    @pl.loop(0, 2 * CH * PADD // 16)
    def _zf(i):
        out_v[pl.ds(i * 16, 16)] = zeros16

    def _reduce(buf):
        @pl.loop(0, CH, unroll=2)
        def _red(j):
            r = j * N

            def _lo(n):
                return lax.bitcast_convert_type(
                    lax.shift_left(rows_v[buf, r + n, :], c16), jnp.float32
                )

            def _hi(n):
                return lax.bitcast_convert_type(
                    rows_v[buf, r + n, :] & cmask, jnp.float32
                )

            ae0, ae1 = _lo(0), _lo(1)
            ao0, ao1 = _hi(0), _hi(1)
            for n in range(2, N, 2):
                ae0 = ae0 + _lo(n)
                ae1 = ae1 + _lo(n + 1)
                ao0 = ao0 + _hi(n)
                ao1 = ao1 + _hi(n + 1)
            base = buf * (CH * PADD) + j * PADD
            out_v[pl.ds(base, 16)] = ae0 + ae1
            out_v[pl.ds(base + 16, 16)] = ao0 + ao1

    def _wait_out(buf):
        pltpu.make_async_copy(
            out_v.at[pl.ds(0, CH * PADD)], word_hbm.at[pl.ds(0, CH * PADD)], sem_o
        ).wait()

    _stage_idx(0, 0).wait()
    _fire_gathers(0)
    _stage_idx(1, 1)

    @pl.loop(0, NCHUNK, step=2)
    def _chunks(c):
        for par in range(2):
            cc = c + par
            nxt = par ^ 1
            _wait_gathers(par)

            @pl.when(cc + 1 < NCHUNK)
            def _():
                _wait_idx(nxt)
                _fire_gathers(nxt)

            @pl.when(cc + 2 < NCHUNK)
            def _():
                _stage_idx(cc + 2, par)

            @pl.when(c >= 2)
            def _():
                _wait_out(par)

            _reduce(par)
            pltpu.async_copy(
                out_v.at[pl.ds(par * CH * PADD, CH * PADD)],
                word_hbm.at[pl.ds((wbase + cc * CH) * PADD, CH * PADD)],
                sem_o,
            )

    _wait_out(0)
    _wait_out(1)


def _gather_sum(x, emb):
    mesh = plsc.VectorSubcoreMesh(
        core_axis_name="c", subcore_axis_name="s", num_cores=NC, num_subcores=NS
    )
    fn = pl.kernel(
        _sc_body,
        out_type=jax.ShapeDtypeStruct((WORDS_SL * PADD,), jnp.float32),
        mesh=mesh,
        scratch_types=[
            pltpu.VMEM((2, ROWS), jnp.int32),
            pltpu.VMEM((2, ROWS, EMB_DIM // 2), jnp.uint32),
            pltpu.VMEM((2 * CH * PADD,), jnp.float32),
            pltpu.SemaphoreType.DMA,
            pltpu.SemaphoreType.DMA,
            pltpu.SemaphoreType.DMA,
        ],
        compiler_params=pltpu.CompilerParams(use_tc_tiling_on_sc=False),
    )
    return fn(x, emb)


def _mm_body(w_ref, ww_ref, b_ref, o_ref):
    o_ref[0] = (
        lax.dot_general(
            ww_ref[...],
            w_ref[0, :, :EMB_DIM],
            dimension_numbers=(((1,), (1,)), ((), ())),
            preferred_element_type=jnp.float32,
        )
        + b_ref[...]
    )


def _project(word3, W, b2, s, carry):
    in_specs = [
        pl.BlockSpec((1, B, PADD), lambda t: (t, 0, 0)),
        pl.BlockSpec((VOCAB, EMB_DIM), lambda t: (0, 0)),
        pl.BlockSpec((VOCAB, 1), lambda t: (0, 0)),
    ]
    args = (word3, W, b2)
    aliases = {}
    if carry is not None:
        in_specs.append(pl.BlockSpec(memory_space=pl.ANY))
        args = (word3, W, b2, carry)
        aliases = {3: 0}

    def body(*refs):
        _mm_body(refs[0], refs[1], refs[2], refs[-1])

    return pl.pallas_call(
        body,
        grid=(TSL,),
        in_specs=in_specs,
        out_specs=pl.BlockSpec((1, VOCAB, B), lambda t, s=s: (s * TSL + t, 0, 0)),
        out_shape=jax.ShapeDtypeStruct((T, VOCAB, B), jnp.float32),
        input_output_aliases=aliases,
        compiler_params=pltpu.CompilerParams(
            dimension_semantics=("arbitrary",)
        ),
    )(*args)


def kernel(x, emb, W, b):
    xf = jnp.transpose(x, (1, 0, 2)).reshape(NUM_WORDS * N)
    embp = jax.lax.bitcast_convert_type(
        emb.astype(jnp.bfloat16).reshape(NUM_EMB, EMB_DIM // 2, 2), jnp.uint32
    )
    Wp = jnp.concatenate([W[:, 0::2], W[:, 1::2]], axis=1)
    b2 = b.reshape(VOCAB, 1)
    out_p = None
    for s in range(NSLICE):
        xf_s = lax.dynamic_slice(xf, (s * WORDS_SL * N,), (WORDS_SL * N,))
        word = _gather_sum(xf_s, embp)
        word3 = word.reshape(TSL, B, PADD)
        out_p = _project(word3, Wp, b2, s, out_p)
    return jnp.transpose(out_p, (2, 0, 1))


# speedup vs baseline: 1.1357x; 1.1357x over previous
import jax
import jax.numpy as jnp
from jax import lax
from jax.experimental import pallas as pl
from jax.experimental.pallas import tpu as pltpu
from jax.experimental.pallas import tpu_sc as plsc

NUM_EMB = 1201
EMB_DIM = 32
VOCAB = 1000
B, T, N = 1024, 50, 20
NUM_WORDS = B * T
NC, NS = 2, 16
NW = NC * NS
NSLICE = 1
TSL = T // NSLICE
WORDS_SL = B * TSL
WPW = WORDS_SL // NW
CH = 80
NCHUNK = WPW // CH
ROWS = CH * N
NGATH = 20
GSZ = ROWS // NGATH
PADD = 128


def _sc_body(xf_hbm, emb_hbm, word_hbm, idx_v, rows_v, out_v, sem, sem_i, sem_o):
    wid = lax.axis_index("s") * NC + lax.axis_index("c")
    wbase = wid * WPW
    c16 = jnp.full((16,), 16, jnp.uint32)
    cmask = jnp.full((16,), 0xFFFF0000, jnp.uint32)

    def _stage_idx(c, buf):
        return pltpu.async_copy(
            xf_hbm.at[pl.ds((wbase + c * CH) * N, ROWS)], idx_v.at[buf], sem_i
        )

    def _wait_idx(buf):
        pltpu.make_async_copy(
            xf_hbm.at[pl.ds(0, ROWS)], idx_v.at[buf], sem_i
        ).wait()

    def _fire_gathers(buf):
        for k in range(NGATH):
            pltpu.async_copy(
                emb_hbm.at[idx_v.at[buf, pl.ds(k * GSZ, GSZ)]],
                rows_v.at[buf, pl.ds(k * GSZ, GSZ)],
                sem,
            )

    def _wait_gathers(buf):
        for k in range(NGATH):
            pltpu.make_async_copy(
                emb_hbm.at[idx_v.at[buf, pl.ds(k * GSZ, GSZ)]],
                rows_v.at[buf, pl.ds(k * GSZ, GSZ)],
                sem,
            ).wait()

    zeros16 = jnp.zeros((16,), jnp.float32)

    @pl.loop(0, 2 * CH * PADD // 16)
    def _zf(i):
        out_v[pl.ds(i * 16, 16)] = zeros16

    def _reduce(buf):
        @pl.loop(0, CH, unroll=2)
        def _red(j):
            r = j * N

            def _lo(n):
                return lax.bitcast_convert_type(
                    lax.shift_left(rows_v[buf, r + n, :], c16), jnp.float32
                )

            def _hi(n):
                return lax.bitcast_convert_type(
                    rows_v[buf, r + n, :] & cmask, jnp.float32
                )

            ae0, ae1 = _lo(0), _lo(1)
            ao0, ao1 = _hi(0), _hi(1)
            for n in range(2, N, 2):
                ae0 = ae0 + _lo(n)
                ae1 = ae1 + _lo(n + 1)
                ao0 = ao0 + _hi(n)
                ao1 = ao1 + _hi(n + 1)
            base = buf * (CH * PADD) + j * PADD
            out_v[pl.ds(base, 16)] = ae0 + ae1
            out_v[pl.ds(base + 16, 16)] = ao0 + ao1

    def _wait_out(buf):
        pltpu.make_async_copy(
            out_v.at[pl.ds(0, CH * PADD)], word_hbm.at[pl.ds(0, CH * PADD)], sem_o
        ).wait()

    _stage_idx(0, 0).wait()
    _fire_gathers(0)
    _stage_idx(1, 1)

    @pl.loop(0, NCHUNK, step=2)
    def _chunks(c):
        for par in range(2):
            cc = c + par
            nxt = par ^ 1
            _wait_gathers(par)

            @pl.when(cc + 1 < NCHUNK)
            def _():
                _wait_idx(nxt)
                _fire_gathers(nxt)

            @pl.when(cc + 2 < NCHUNK)
            def _():
                _stage_idx(cc + 2, par)

            @pl.when(c >= 2)
            def _():
                _wait_out(par)

            _reduce(par)
            pltpu.async_copy(
                out_v.at[pl.ds(par * CH * PADD, CH * PADD)],
                word_hbm.at[pl.ds((wbase + cc * CH) * PADD, CH * PADD)],
                sem_o,
            )

    _wait_out(0)
    _wait_out(1)


def _gather_sum(x, emb):
    mesh = plsc.VectorSubcoreMesh(
        core_axis_name="c", subcore_axis_name="s", num_cores=NC, num_subcores=NS
    )
    fn = pl.kernel(
        _sc_body,
        out_type=jax.ShapeDtypeStruct((WORDS_SL * PADD,), jnp.float32),
        mesh=mesh,
        scratch_types=[
            pltpu.VMEM((2, ROWS), jnp.int32),
            pltpu.VMEM((2, ROWS, EMB_DIM // 2), jnp.uint32),
            pltpu.VMEM((2 * CH * PADD,), jnp.float32),
            pltpu.SemaphoreType.DMA,
            pltpu.SemaphoreType.DMA,
            pltpu.SemaphoreType.DMA,
        ],
        compiler_params=pltpu.CompilerParams(use_tc_tiling_on_sc=False),
    )
    return fn(x, emb)


def _mm_body(w_ref, ww_ref, b_ref, o_ref):
    o_ref[0] = (
        lax.dot_general(
            ww_ref[...],
            w_ref[0, :, :EMB_DIM],
            dimension_numbers=(((1,), (1,)), ((), ())),
            preferred_element_type=jnp.float32,
        )
        + b_ref[...]
    )


def _project(word3, W, b2, s, carry):
    in_specs = [
        pl.BlockSpec((1, B, PADD), lambda t: (t, 0, 0)),
        pl.BlockSpec((VOCAB, EMB_DIM), lambda t: (0, 0)),
        pl.BlockSpec((VOCAB, 1), lambda t: (0, 0)),
    ]
    args = (word3, W, b2)
    aliases = {}
    if carry is not None:
        in_specs.append(pl.BlockSpec(memory_space=pl.ANY))
        args = (word3, W, b2, carry)
        aliases = {3: 0}

    def body(*refs):
        _mm_body(refs[0], refs[1], refs[2], refs[-1])

    return pl.pallas_call(
        body,
        grid=(TSL,),
        in_specs=in_specs,
        out_specs=pl.BlockSpec((1, VOCAB, B), lambda t, s=s: (s * TSL + t, 0, 0)),
        out_shape=jax.ShapeDtypeStruct((T, VOCAB, B), jnp.float32),
        input_output_aliases=aliases,
        compiler_params=pltpu.CompilerParams(
            dimension_semantics=("arbitrary",)
        ),
    )(*args)


def kernel(x, emb, W, b):
    xf = jnp.transpose(x, (1, 0, 2)).reshape(NUM_WORDS * N)
    embp = jax.lax.bitcast_convert_type(
        emb.astype(jnp.bfloat16).reshape(NUM_EMB, EMB_DIM // 2, 2), jnp.uint32
    )
    Wp = jnp.concatenate([W[:, 0::2], W[:, 1::2]], axis=1)
    b2 = b.reshape(VOCAB, 1)
    out_p = None
    for s in range(NSLICE):
        xf_s = lax.dynamic_slice(xf, (s * WORDS_SL * N,), (WORDS_SL * N,))
        word = _gather_sum(xf_s, embp)
        word3 = word.reshape(TSL, B, PADD)
        out_p = _project(word3, Wp, b2, s, out_p)
    return jnp.transpose(out_p, (2, 0, 1))


# speedup vs baseline: 1.1481x; 1.0109x over previous
import jax
import jax.numpy as jnp
from jax import lax
from jax.experimental import pallas as pl
from jax.experimental.pallas import tpu as pltpu
from jax.experimental.pallas import tpu_sc as plsc

NUM_EMB = 1201
EMB_DIM = 32
VOCAB = 1000
B, T, N = 1024, 50, 20
NUM_WORDS = B * T
NC, NS = 2, 16
NW = NC * NS
NSLICE = 1
TSL = T // NSLICE
WORDS_SL = B * TSL
WPW = WORDS_SL // NW
CH = 32
NCHUNK = WPW // CH
ROWS = CH * N
NGATH = 5
GSZ = ROWS // NGATH
PADD = 128


def _sc_body(xf_hbm, emb_hbm, word_hbm, idx_v, rows_v, out_v, sem, sem_i, sem_o):
    wid = lax.axis_index("s") * NC + lax.axis_index("c")
    wbase = wid * WPW
    c16 = jnp.full((16,), 16, jnp.uint32)
    cmask = jnp.full((16,), 0xFFFF0000, jnp.uint32)

    def _stage_idx(c, buf):
        return pltpu.async_copy(
            xf_hbm.at[pl.ds((wbase + c * CH) * N, ROWS)], idx_v.at[buf], sem_i
        )

    def _wait_idx(buf):
        pltpu.make_async_copy(
            xf_hbm.at[pl.ds(0, ROWS)], idx_v.at[buf], sem_i
        ).wait()

    def _fire_gathers(buf):
        for k in range(NGATH):
            pltpu.async_copy(
                emb_hbm.at[idx_v.at[buf, pl.ds(k * GSZ, GSZ)]],
                rows_v.at[buf, pl.ds(k * GSZ, GSZ)],
                sem,
            )

    def _wait_gathers(buf):
        for k in range(NGATH):
            pltpu.make_async_copy(
                emb_hbm.at[idx_v.at[buf, pl.ds(k * GSZ, GSZ)]],
                rows_v.at[buf, pl.ds(k * GSZ, GSZ)],
                sem,
            ).wait()

    zeros16 = jnp.zeros((16,), jnp.float32)

    @pl.loop(0, 2 * CH * PADD // 16)
    def _zf(i):
        out_v[pl.ds(i * 16, 16)] = zeros16

    def _reduce(buf):
        @pl.loop(0, CH, unroll=2)
        def _red(j):
            r = j * N

            def _lo(n):
                return lax.bitcast_convert_type(
                    lax.shift_left(rows_v[buf, r + n, :], c16), jnp.float32
                )

            def _hi(n):
                return lax.bitcast_convert_type(
                    rows_v[buf, r + n, :] & cmask, jnp.float32
                )

            ae0, ae1 = _lo(0), _lo(1)
            ao0, ao1 = _hi(0), _hi(1)
            for n in range(2, N, 2):
                ae0 = ae0 + _lo(n)
                ae1 = ae1 + _lo(n + 1)
                ao0 = ao0 + _hi(n)
                ao1 = ao1 + _hi(n + 1)
            base = buf * (CH * PADD) + j * PADD
            out_v[pl.ds(base, 16)] = ae0 + ae1
            out_v[pl.ds(base + 16, 16)] = ao0 + ao1

    def _wait_out(buf):
        pltpu.make_async_copy(
            out_v.at[pl.ds(0, CH * PADD)], word_hbm.at[pl.ds(0, CH * PADD)], sem_o
        ).wait()

    _stage_idx(0, 0).wait()
    _fire_gathers(0)
    _stage_idx(1, 1)

    @pl.loop(0, NCHUNK, step=2)
    def _chunks(c):
        for par in range(2):
            cc = c + par
            nxt = par ^ 1
            _wait_gathers(par)

            @pl.when(cc + 1 < NCHUNK)
            def _():
                _wait_idx(nxt)
                _fire_gathers(nxt)

            @pl.when(cc + 2 < NCHUNK)
            def _():
                _stage_idx(cc + 2, par)

            @pl.when(c >= 2)
            def _():
                _wait_out(par)

            _reduce(par)
            pltpu.async_copy(
                out_v.at[pl.ds(par * CH * PADD, CH * PADD)],
                word_hbm.at[pl.ds((wbase + cc * CH) * PADD, CH * PADD)],
                sem_o,
            )

    _wait_out(0)
    _wait_out(1)


def _gather_sum(x, emb):
    mesh = plsc.VectorSubcoreMesh(
        core_axis_name="c", subcore_axis_name="s", num_cores=NC, num_subcores=NS
    )
    fn = pl.kernel(
        _sc_body,
        out_type=jax.ShapeDtypeStruct((WORDS_SL * PADD,), jnp.float32),
        mesh=mesh,
        scratch_types=[
            pltpu.VMEM((2, ROWS), jnp.int32),
            pltpu.VMEM((2, ROWS, EMB_DIM // 2), jnp.uint32),
            pltpu.VMEM((2 * CH * PADD,), jnp.float32),
            pltpu.SemaphoreType.DMA,
            pltpu.SemaphoreType.DMA,
            pltpu.SemaphoreType.DMA,
        ],
        compiler_params=pltpu.CompilerParams(use_tc_tiling_on_sc=False),
    )
    return fn(x, emb)


def _mm_body(w_ref, ww_ref, b_ref, o_ref):
    o_ref[0] = (
        lax.dot_general(
            ww_ref[...],
            w_ref[0, :, :EMB_DIM],
            dimension_numbers=(((1,), (1,)), ((), ())),
            preferred_element_type=jnp.float32,
        )
        + b_ref[...]
    )


def _project(word3, W, b2, s, carry):
    in_specs = [
        pl.BlockSpec((1, B, PADD), lambda t: (t, 0, 0)),
        pl.BlockSpec((VOCAB, EMB_DIM), lambda t: (0, 0)),
        pl.BlockSpec((VOCAB, 1), lambda t: (0, 0)),
    ]
    args = (word3, W, b2)
    aliases = {}
    if carry is not None:
        in_specs.append(pl.BlockSpec(memory_space=pl.ANY))
        args = (word3, W, b2, carry)
        aliases = {3: 0}

    def body(*refs):
        _mm_body(refs[0], refs[1], refs[2], refs[-1])

    return pl.pallas_call(
        body,
        grid=(TSL,),
        in_specs=in_specs,
        out_specs=pl.BlockSpec((1, VOCAB, B), lambda t, s=s: (s * TSL + t, 0, 0)),
        out_shape=jax.ShapeDtypeStruct((T, VOCAB, B), jnp.float32),
        input_output_aliases=aliases,
        compiler_params=pltpu.CompilerParams(
            dimension_semantics=("arbitrary",)
        ),
    )(*args)


def kernel(x, emb, W, b):
    xf = jnp.transpose(x, (1, 0, 2)).reshape(NUM_WORDS * N)
    embp = jax.lax.bitcast_convert_type(
        emb.astype(jnp.bfloat16).reshape(NUM_EMB, EMB_DIM // 2, 2), jnp.uint32
    )
    Wp = jnp.concatenate([W[:, 0::2], W[:, 1::2]], axis=1)
    b2 = b.reshape(VOCAB, 1)
    out_p = None
    for s in range(NSLICE):
        xf_s = lax.dynamic_slice(xf, (s * WORDS_SL * N,), (WORDS_SL * N,))
        word = _gather_sum(xf_s, embp)
        word3 = word.reshape(TSL, B, PADD)
        out_p = _project(word3, Wp, b2, s, out_p)
    return jnp.transpose(out_p, (2, 0, 1))
